# baseline (speedup 1.0000x reference)
import jax
import jax.numpy as jnp
from jax import lax
from jax.experimental import pallas as pl

N, E, D, H, OUT = 10000, 320000, 128, 256, 10
EPS = 1e-05


def _leaky(x):
    return jnp.where(x >= 0, x, 0.01 * x)


def _identity_body(x_ref, o_ref):
    o_ref[...] = x_ref[...]


_identity = pl.pallas_call(
    _identity_body, out_shape=jax.ShapeDtypeStruct((N, D), jnp.float32))


def _mm_body(a_ref, b_ref, o_ref):
    o_ref[...] = jax.lax.dot_general(
        a_ref[...], b_ref[...], (((1,), (0,)), ((), ())),
        precision=jax.lax.Precision.HIGHEST,
        preferred_element_type=jnp.float32)


def _mm(a, b):
    return pl.pallas_call(
        _mm_body,
        out_shape=jax.ShapeDtypeStruct((a.shape[0], b.shape[1]), jnp.float32),
    )(a, b)


def _seq_scatter(msg, dst, width):
    def body(e, acc):
        return lax.dynamic_update_slice(
            acc,
            lax.dynamic_slice(acc, (dst[e], 0), (1, width)) + msg[e][None, :],
            (dst[e], 0))
    return lax.fori_loop(0, E, body, jnp.zeros((N, width), msg.dtype))


def _graph_conv(x, W, src, dst):
    deg_out = jnp.clip(jnp.bincount(src, length=N), 1, None).astype(x.dtype)
    deg_in = jnp.clip(jnp.bincount(dst, length=N), 1, None).astype(x.dtype)
    h = x * (deg_out ** -0.5)[:, None]
    msg = jnp.take(h, src, axis=0)
    agg = jnp.zeros((N, x.shape[1]), x.dtype).at[dst].add(msg)
    agg = agg * (deg_in ** -0.5)[:, None]
    return _mm(agg, W)


def _graph_norm(x, alpha, gamma, beta):
    mean = jnp.mean(x, axis=0, keepdims=True)
    sub = x - alpha[None, :] * mean
    var = jnp.mean(sub * sub, axis=0, keepdims=True)
    return gamma[None, :] * sub / jnp.sqrt(var + EPS) + beta[None, :]


def kernel(features, edge_index, W1, W2, gn1_alpha, gn1_gamma, gn1_beta,
           gn2_alpha, gn2_gamma, gn2_beta, Wl, bl, Wc):
    src, dst = edge_index[0], edge_index[1]
    features = _identity(features)
    h = _graph_conv(features, W1, src, dst)
    h = _leaky(h)
    h = _graph_norm(h, gn1_alpha, gn1_gamma, gn1_beta)
    h = _graph_conv(h, W2, src, dst)
    h = _leaky(h)
    h = _graph_norm(h, gn2_alpha, gn2_gamma, gn2_beta)
    pooled = jnp.mean(h, axis=0, keepdims=True)
    y = pooled @ Wl.T + bl[None, :]
    y = _leaky(y)
    m = jnp.mean(y, axis=-1, keepdims=True)
    v = jnp.mean((y - m) ** 2, axis=-1, keepdims=True)
    y = (y - m) / jnp.sqrt(v + EPS)
    return y @ Wc.T
